# Initial kernel scaffold; baseline (speedup 1.0000x reference)
import jax
import jax.numpy as jnp
from jax.experimental import pallas as pl


def kernel(graph_edge_index, graph_edge_vals, embed):
    raise NotImplementedError("write your pallas kernel here")


# speedup vs baseline: 1.0302x; 1.0302x over previous
import functools
import jax
import jax.numpy as jnp
from jax.experimental import pallas as pl
from jax.experimental.pallas import tpu as pltpu

_N = 4096
_D = 64
_B = 512
_KB = 512
_NI = _N // _B
_NK = _N // _KB
_SB = 128


def _softmax_emb(strip, embed):
    mask = strip != 0.0
    neg = jnp.where(mask, strip, -jnp.inf)
    rowmax = jnp.max(neg, axis=1, keepdims=True)
    rowmax = jnp.where(jnp.isfinite(rowmax), rowmax, 0.0)
    e = jnp.where(mask, jnp.exp(strip - rowmax), 0.0)
    denom = jnp.sum(e, axis=1, keepdims=True)
    s = e / jnp.where(denom == 0.0, 1.0, denom)
    return jax.lax.dot_general(
        s, embed, (((1,), (0,)), ((), ())), preferred_element_type=jnp.float32
    )


def _gcn_body(lhs_ref, rhs_ref, emb_ref, e1_ref, e2_ref, e3_ref, light_ref,
              a1_ref, a2_ref, a3_ref):
    i = pl.program_id(0)
    p = pl.program_id(1)
    k = pl.program_id(2)

    @pl.when(jnp.logical_and(p == 0, k == 0))
    def _():
        a2_ref[...] = jnp.zeros_like(a2_ref)

    @pl.when(jnp.logical_and(p == 1, k == 0))
    def _():
        a3_ref[...] = jnp.zeros_like(a3_ref)

    @pl.when(p == 0)
    def _():
        lhs = lhs_ref[...]
        a1_ref[:, pl.ds(k * _KB, _KB)] = lhs
        a2_ref[...] += jax.lax.dot_general(
            lhs, rhs_ref[...], (((1,), (0,)), ((), ())),
            preferred_element_type=jnp.float32,
        )

    @pl.when(p == 1)
    def _():
        lhs2 = a2_ref[:, pl.ds(k * _KB, _KB)]
        a3_ref[...] += jax.lax.dot_general(
            lhs2, rhs_ref[...], (((1,), (0,)), ((), ())),
            preferred_element_type=jnp.float32,
        )

    @pl.when(jnp.logical_and(p == 1, k == _NK - 1))
    def _():
        emb = emb_ref[...]
        for c in range(0, _B, _SB):
            sl = pl.ds(c, _SB)
            e1 = _softmax_emb(a1_ref[sl, :], emb)
            e2 = _softmax_emb(a2_ref[sl, :], emb)
            e3 = _softmax_emb(a3_ref[sl, :], emb)
            e1_ref[sl, :] = e1
            e2_ref[sl, :] = e2
            e3_ref[sl, :] = e3
            my_emb = emb_ref[pl.ds(i * _B + c, _SB), :]
            light_ref[sl, :] = (my_emb + e1 + e2 + e3) * 0.25


def _gcn_call(a, embed):
    out = jax.ShapeDtypeStruct((_N, _D), jnp.float32)
    e1, e2, e3, light = pl.pallas_call(
        _gcn_body,
        grid=(_NI, 2, _NK),
        in_specs=[
            pl.BlockSpec((_B, _KB), lambda i, p, k: (i, k * (1 - p))),
            pl.BlockSpec((_KB, _N), lambda i, p, k: (k, 0)),
            pl.BlockSpec((_N, _D), lambda i, p, k: (0, 0)),
        ],
        out_specs=[pl.BlockSpec((_B, _D), lambda i, p, k: (i, 0))] * 4,
        out_shape=[out] * 4,
        scratch_shapes=[pltpu.VMEM((_B, _N), jnp.float32)] * 3,
    )(a, a, embed)
    return e1, e2, e3, light


def kernel(graph_edge_index, graph_edge_vals, embed):
    a = jnp.zeros((_N, _N), jnp.float32).at[
        graph_edge_index[0], graph_edge_index[1]
    ].add(graph_edge_vals)
    e1, e2, e3, light = _gcn_call(a, embed)
    return (light, (embed, e1, e2, e3))
